# baseline (speedup 1.0000x reference)
import functools

import jax
import jax.numpy as jnp
from jax import lax
from jax.experimental import pallas as pl
from jax.experimental.pallas import tpu as pltpu
from jax.experimental.pallas import tpu_sc as plsc

VOCAB = 1000000
EMBED_DIM = 64
BATCH = 4096
SEQ = 200

NC = 2
NS = 16
NW = NC * NS
RPW = BATCH // NW

CHUNK_A = 128
CHUNK_B = SEQ - CHUNK_A

TB = 16384
NBLK = 31
SPLIT = NBLK * TB
LAST_IN_BLK = (VOCAB + TB - 1) // TB - 1


def _pack_kernel(lo_ref, hi_ref, o_ref):
    ident = jnp.eye(EMBED_DIM, dtype=jnp.float32)
    dn = (((0,), (0,)), ((), ()))
    o_ref[:, :EMBED_DIM] = lax.dot_general(
        lo_ref[...], ident, dn,
        preferred_element_type=jnp.float32).astype(jnp.bfloat16)
    o_ref[:, EMBED_DIM:] = lax.dot_general(
        hi_ref[...], ident, dn,
        preferred_element_type=jnp.float32).astype(jnp.bfloat16)


def _pack_pairs(table_t):
    return pl.pallas_call(
        _pack_kernel,
        grid=(NBLK,),
        in_specs=[
            pl.BlockSpec((EMBED_DIM, TB), lambda g: (0, g)),
            pl.BlockSpec(
                (EMBED_DIM, TB),
                lambda g: (0, jnp.minimum(g + NBLK, LAST_IN_BLK))),
        ],
        out_specs=pl.BlockSpec((TB, 2 * EMBED_DIM), lambda g: (g, 0)),
        out_shape=jax.ShapeDtypeStruct((SPLIT, 2 * EMBED_DIM), jnp.bfloat16),
    )(table_t, table_t)


def _encoder_kernel(ids_hbm, table_hbm, out_hbm,
                    idx_all, rows0, rows1, out_v, sem0, sem1):
    wid = lax.axis_index("s") * NC + lax.axis_index("c")
    base = wid * RPW

    inv = jnp.float32(1.0 / SEQ)
    rows = (rows0, rows1)
    sems = (sem0, sem1)

    pltpu.sync_copy(ids_hbm.at[pl.ds(base * SEQ, RPW * SEQ)], idx_all)

    def fire(r, slot):
        off = r * SEQ
        pltpu.async_copy(
            table_hbm.at[idx_all.at[pl.ds(off, CHUNK_A)]],
            rows[slot].at[pl.ds(0, CHUNK_A)], sems[slot])
        pltpu.async_copy(
            table_hbm.at[idx_all.at[pl.ds(off + CHUNK_A, CHUNK_B)]],
            rows[slot].at[pl.ds(CHUNK_A, CHUNK_B)], sems[slot])

    def wait(slot):
        pltpu.make_async_copy(
            table_hbm.at[idx_all.at[pl.ds(0, CHUNK_A)]],
            rows[slot].at[pl.ds(0, CHUNK_A)], sems[slot]).wait()
        pltpu.make_async_copy(
            table_hbm.at[idx_all.at[pl.ds(0, CHUNK_B)]],
            rows[slot].at[pl.ds(CHUNK_A, CHUNK_B)], sems[slot]).wait()

    def accum(r, slot):
        buf = rows[slot]

        def acc_body(j, accs):
            out = []
            for c in range(2):
                x = buf[j, pl.ds(32 * c, 32)]
                a, b = plsc.unpack(x, format=plsc.PackFormat.INTERLEAVED)
                out.append(accs[2 * c] + a)
                out.append(accs[2 * c + 1] + b)
            return tuple(out)

        zeros = tuple(jnp.zeros((16,), jnp.float32) for _ in range(4))
        accs = lax.fori_loop(0, SEQ, acc_body, zeros, unroll=8)
        for k in range(4):
            out_v[r, pl.ds(16 * k, 16)] = accs[k] * inv

    fire(0, 0)

    def outer(rr, carry):
        r0 = 2 * rr
        fire(r0 + 1, 1)
        wait(0)
        accum(r0, 0)

        @pl.when(r0 + 2 < RPW)
        def _():
            fire(r0 + 2, 0)

        wait(1)
        accum(r0 + 1, 1)
        return carry

    lax.fori_loop(0, RPW // 2, outer, 0)
    pltpu.sync_copy(out_v, out_hbm.at[pl.ds(base, RPW)])


def kernel(text_ids, table):
    ids = text_ids.astype(jnp.int32)
    rows_flat = jnp.where(
        ids < SPLIT, 2 * ids, 2 * (ids - SPLIT) + 1).reshape(-1)
    table2 = _pack_pairs(table.T).reshape(2 * SPLIT, EMBED_DIM)
    inv_perm = jnp.asarray(
        [32 * (f // 32) + (f % 32) // 2 + 16 * (f % 2) for f in range(64)],
        dtype=jnp.int32)
    mesh = plsc.VectorSubcoreMesh(core_axis_name="c", subcore_axis_name="s")
    k = functools.partial(
        pl.kernel,
        mesh=mesh,
        out_type=jax.ShapeDtypeStruct((BATCH, EMBED_DIM), jnp.float32),
        scratch_types=[
            pltpu.VMEM((RPW * SEQ,), jnp.int32),
            pltpu.VMEM((SEQ, EMBED_DIM), jnp.bfloat16),
            pltpu.VMEM((SEQ, EMBED_DIM), jnp.bfloat16),
            pltpu.VMEM((RPW, EMBED_DIM), jnp.float32),
            pltpu.SemaphoreType.DMA,
            pltpu.SemaphoreType.DMA,
        ],
        compiler_params=pltpu.CompilerParams(
            use_tc_tiling_on_sc=False, needs_layout_passes=False),
    )(_encoder_kernel)
    return k(rows_flat, table2)[:, inv_perm]


# speedup vs baseline: 2.0692x; 2.0692x over previous
import functools

import jax
import jax.numpy as jnp
from jax import lax
from jax.experimental import pallas as pl
from jax.experimental.pallas import tpu as pltpu
from jax.experimental.pallas import tpu_sc as plsc

VOCAB = 1000000
EMBED_DIM = 64
BATCH = 4096
SEQ = 200

NC = 2
NS = 16
NW = NC * NS
RPW = BATCH // NW

CHUNK_A = 128
CHUNK_B = SEQ - CHUNK_A

TB = 16384
NBLK = 31
SPLIT = NBLK * TB
LAST_IN_BLK = (VOCAB + TB - 1) // TB - 1


def _pack_kernel(lo_ref, hi_ref, o_ref):
    ident = jnp.eye(EMBED_DIM, dtype=jnp.float32)
    dn = (((0,), (0,)), ((), ()))
    o_ref[:, :EMBED_DIM] = lax.dot_general(
        lo_ref[...], ident, dn, preferred_element_type=jnp.float32)
    o_ref[:, EMBED_DIM:] = lax.dot_general(
        hi_ref[...], ident, dn, preferred_element_type=jnp.float32)


def _pack_pairs(table_t):
    return pl.pallas_call(
        _pack_kernel,
        grid=(NBLK,),
        in_specs=[
            pl.BlockSpec((EMBED_DIM, TB), lambda g: (0, g)),
            pl.BlockSpec(
                (EMBED_DIM, TB),
                lambda g: (0, jnp.minimum(g + NBLK, LAST_IN_BLK))),
        ],
        out_specs=pl.BlockSpec((TB, 2 * EMBED_DIM), lambda g: (g, 0)),
        out_shape=jax.ShapeDtypeStruct((SPLIT, 2 * EMBED_DIM), jnp.float32),
    )(table_t, table_t)


def _encoder_kernel(ids_hbm, table_hbm, out_hbm,
                    idx_all, rows0, rows1, out_v, sem0, sem1):
    wid = lax.axis_index("s") * NC + lax.axis_index("c")
    base = wid * RPW

    inv = jnp.float32(1.0 / SEQ)
    rows = (rows0, rows1)
    sems = (sem0, sem1)

    pltpu.sync_copy(ids_hbm.at[pl.ds(base * SEQ, RPW * SEQ)], idx_all)

    def fire(r, slot):
        off = r * SEQ
        pltpu.async_copy(
            table_hbm.at[idx_all.at[pl.ds(off, CHUNK_A)]],
            rows[slot].at[pl.ds(0, CHUNK_A)], sems[slot])
        pltpu.async_copy(
            table_hbm.at[idx_all.at[pl.ds(off + CHUNK_A, CHUNK_B)]],
            rows[slot].at[pl.ds(CHUNK_A, CHUNK_B)], sems[slot])

    def wait(slot):
        pltpu.make_async_copy(
            table_hbm.at[idx_all.at[pl.ds(0, CHUNK_A)]],
            rows[slot].at[pl.ds(0, CHUNK_A)], sems[slot]).wait()
        pltpu.make_async_copy(
            table_hbm.at[idx_all.at[pl.ds(0, CHUNK_B)]],
            rows[slot].at[pl.ds(CHUNK_A, CHUNK_B)], sems[slot]).wait()

    def accum(r, slot):
        buf = rows[slot]

        def acc_body(j, accs):
            return tuple(
                accs[k] + buf[j, pl.ds(16 * k, 16)] for k in range(4))

        zeros = tuple(jnp.zeros((16,), jnp.float32) for _ in range(4))
        accs = lax.fori_loop(0, SEQ, acc_body, zeros, unroll=8)
        for k in range(4):
            out_v[r, pl.ds(16 * k, 16)] = accs[k] * inv

    fire(0, 0)

    def outer(rr, carry):
        r0 = 2 * rr
        fire(r0 + 1, 1)
        wait(0)
        accum(r0, 0)

        @pl.when(r0 + 2 < RPW)
        def _():
            fire(r0 + 2, 0)

        wait(1)
        accum(r0 + 1, 1)
        return carry

    lax.fori_loop(0, RPW // 2, outer, 0)
    pltpu.sync_copy(out_v, out_hbm.at[pl.ds(base, RPW)])


def kernel(text_ids, table):
    ids = text_ids.astype(jnp.int32)
    rows_flat = jnp.where(
        ids < SPLIT, 2 * ids, 2 * (ids - SPLIT) + 1).reshape(-1)
    table2 = _pack_pairs(table.T).reshape(2 * SPLIT, EMBED_DIM)
    mesh = plsc.VectorSubcoreMesh(core_axis_name="c", subcore_axis_name="s")
    k = functools.partial(
        pl.kernel,
        mesh=mesh,
        out_type=jax.ShapeDtypeStruct((BATCH, EMBED_DIM), jnp.float32),
        scratch_types=[
            pltpu.VMEM((RPW * SEQ,), jnp.int32),
            pltpu.VMEM((SEQ, EMBED_DIM), jnp.float32),
            pltpu.VMEM((SEQ, EMBED_DIM), jnp.float32),
            pltpu.VMEM((RPW, EMBED_DIM), jnp.float32),
            pltpu.SemaphoreType.DMA,
            pltpu.SemaphoreType.DMA,
        ],
        compiler_params=pltpu.CompilerParams(use_tc_tiling_on_sc=False),
    )(_encoder_kernel)
    return k(rows_flat, table2)
